# Initial kernel scaffold; baseline (speedup 1.0000x reference)
import jax
import jax.numpy as jnp
from jax.experimental import pallas as pl


def kernel(input_ids, table):
    raise NotImplementedError("write your pallas kernel here")


# speedup vs baseline: 9.2299x; 9.2299x over previous
import functools

import jax
import jax.numpy as jnp
from jax import lax
from jax.experimental import pallas as pl
from jax.experimental.pallas import tpu as pltpu
from jax.experimental.pallas import tpu_sc as plsc

_D = 128
_CHUNK = 128


def _embed(ids2d, table):
    n_rows = ids2d.shape[0] * ids2d.shape[1]
    info = plsc.get_sparse_core_info()
    nc = info.num_cores
    nw = nc * info.num_subcores
    rows_w = n_rows // nw
    nch = rows_w // _CHUNK

    mesh = plsc.VectorSubcoreMesh(core_axis_name="c", subcore_axis_name="s")

    @functools.partial(
        pl.kernel,
        mesh=mesh,
        out_type=jax.ShapeDtypeStruct((n_rows, _D), jnp.float32),
        scratch_types=[
            pltpu.VMEM((nch, _CHUNK), jnp.int32),
            pltpu.VMEM((_CHUNK, _D), jnp.float32),
            pltpu.VMEM((_CHUNK, _D), jnp.float32),
            pltpu.SemaphoreType.DMA,
            pltpu.SemaphoreType.DMA,
            pltpu.SemaphoreType.DMA,
            pltpu.SemaphoreType.DMA,
        ],
    )
    def emb(ids_hbm, table_hbm, out_hbm, idx_v, buf_a, buf_b, ga, gb, sa, sb):
        wid = lax.axis_index("s") * nc + lax.axis_index("c")
        row0 = wid * rows_w
        pltpu.sync_copy(ids_hbm.at[pl.ds(wid * nch, nch)], idx_v)

        def gather_start(g, buf, sem):
            pltpu.async_copy(table_hbm.at[idx_v.at[g]], buf, sem)

        def gather_wait(g, buf, sem):
            pltpu.make_async_copy(table_hbm.at[idx_v.at[g]], buf, sem).wait()

        def store_start(g, buf, sem):
            pltpu.async_copy(
                buf, out_hbm.at[pl.ds(row0 + g * _CHUNK, _CHUNK)], sem)

        def store_wait(g, buf, sem):
            pltpu.make_async_copy(
                buf, out_hbm.at[pl.ds(row0 + g * _CHUNK, _CHUNK)], sem).wait()

        gather_start(0, buf_a, ga)
        gather_start(1, buf_b, gb)

        def body(i, carry):
            g = 2 * i
            gather_wait(g, buf_a, ga)
            store_start(g, buf_a, sa)
            store_wait(g, buf_a, sa)
            gather_start(g + 2, buf_a, ga)
            gather_wait(g + 1, buf_b, gb)
            store_start(g + 1, buf_b, sb)
            store_wait(g + 1, buf_b, sb)
            gather_start(g + 3, buf_b, gb)
            return carry

        lax.fori_loop(0, nch // 2 - 1, body, 0)

        g_last = nch - 2
        gather_wait(g_last, buf_a, ga)
        store_start(g_last, buf_a, sa)
        gather_wait(g_last + 1, buf_b, gb)
        store_start(g_last + 1, buf_b, sb)
        store_wait(g_last, buf_a, sa)
        store_wait(g_last + 1, buf_b, sb)

    return emb(ids2d, table)


def kernel(input_ids, table):
    b, l = input_ids.shape
    n = b * l
    ids2d = input_ids.astype(jnp.int32).reshape(n // _CHUNK, _CHUNK)
    out = _embed(ids2d, table)
    return out.reshape(b, l, _D)
